# baseline (speedup 1.0000x reference)
import functools

import jax
import jax.numpy as jnp
from jax import lax
from jax.experimental import pallas as pl
from jax.experimental.pallas import tpu as pltpu
from jax.experimental.pallas import tpu_sc as plsc

B = 128
L = 512
D = 512
V = 42
N = B * L
NC, NS = 2, 16
NW = NC * NS
TPW = N // NW
C = 32
NCHUNK = TPW // C
NBODY = NCHUNK // 2
LANES = 16
G = D // LANES


def _positional_encoding(max_len, d_model):
    even_i = jnp.arange(0, d_model, 2).astype(jnp.float32)
    denominator = jnp.power(10000.0, even_i / d_model)
    position = jnp.arange(max_len, dtype=jnp.float32).reshape(max_len, 1)
    even_pe = jnp.sin(position / denominator)
    odd_pe = jnp.cos(position / denominator)
    return jnp.stack([even_pe, odd_pe], axis=2).reshape(max_len, d_model)


@functools.partial(
    pl.kernel,
    mesh=plsc.VectorSubcoreMesh(core_axis_name="c", subcore_axis_name="s"),
    out_type=jax.ShapeDtypeStruct((N, D), jnp.float32),
    scratch_types=[
        pltpu.VMEM((V * D,), jnp.float32),
        pltpu.VMEM((C,), jnp.int32),
        pltpu.VMEM((C,), jnp.int32),
        pltpu.VMEM((C, D), jnp.float32),
        pltpu.VMEM((C, D), jnp.float32),
        pltpu.SemaphoreType.DMA,
        pltpu.SemaphoreType.DMA,
        pltpu.SemaphoreType.DMA,
        pltpu.SemaphoreType.DMA,
        pltpu.SemaphoreType.DMA,
        pltpu.SemaphoreType.DMA,
    ],
)
def _emb_pe(table_hbm, idx_hbm, pe_hbm, out_hbm,
            table_v, ids0, ids1, rows0, rows1,
            si0, si1, sp0, sp1, so0, so1):
    wid = lax.axis_index("s") * NC + lax.axis_index("c")
    tok_base = wid * TPW

    ids = (ids0, ids1)
    rows = (rows0, rows1)
    si = (si0, si1)
    sp = (sp0, sp1)
    so = (so0, so1)

    pltpu.sync_copy(table_hbm, table_v)

    def issue_idx(c, p):
        return pltpu.async_copy(idx_hbm.at[wid, c], ids[p], si[p])

    def wait_idx(p):
        pltpu.make_async_copy(idx_hbm.at[wid, 0], ids[p], si[p]).wait()

    def issue_pe(c, p):
        l0 = (c % (L // C)) * C
        return pltpu.async_copy(pe_hbm.at[pl.ds(l0, C)], rows[p], sp[p])

    def add_table_rows(p):
        r, s = rows[p], ids[p]

        def blk16(i, acc):
            t0 = i * LANES
            offvec = s[pl.ds(t0, LANES)] * D
            bases = [offvec[j] for j in range(LANES)]
            for j in range(LANES):
                base = bases[j]
                batches = [
                    [(g0 + jj) * LANES for jj in range(8)]
                    for g0 in range(0, G, 8)
                ]
                prev_offs = batches[0]
                prev_vals = [
                    table_v[pl.ds(base + o, LANES)] for o in prev_offs
                ]
                for bt in batches[1:]:
                    cur_vals = [
                        table_v[pl.ds(base + o, LANES)] for o in bt
                    ]
                    for o, val in zip(prev_offs, prev_vals):
                        plsc.addupdate(r.at[t0 + j, pl.ds(o, LANES)], val)
                    prev_offs, prev_vals = bt, cur_vals
                for o, val in zip(prev_offs, prev_vals):
                    plsc.addupdate(r.at[t0 + j, pl.ds(o, LANES)], val)
            return acc

        lax.fori_loop(0, C // LANES, blk16, 0)

    def issue_out(c, p):
        return pltpu.async_copy(
            rows[p], out_hbm.at[pl.ds(tok_base + c * C, C)], so[p]
        )

    def wait_out(p):
        pltpu.make_async_copy(
            rows[p], out_hbm.at[pl.ds(tok_base, C)], so[p]
        ).wait()

    issue_idx(0, 0)
    issue_idx(1, 1)

    def body(k, carry):
        c0 = 2 * k
        c1 = 2 * k + 1

        @pl.when(k > 0)
        def _():
            wait_out(0)
            wait_out(1)

        q0 = issue_pe(c0, 0)
        q1 = issue_pe(c1, 1)

        wait_idx(0)
        q0.wait()
        add_table_rows(0)

        @pl.when(k < NBODY - 1)
        def _():
            issue_idx(c0 + 2, 0)

        issue_out(c0, 0)

        wait_idx(1)
        q1.wait()
        add_table_rows(1)

        @pl.when(k < NBODY - 1)
        def _():
            issue_idx(c1 + 2, 1)

        issue_out(c1, 1)
        return carry

    lax.fori_loop(0, NBODY, body, 0)
    wait_out(0)
    wait_out(1)


def kernel(x, table, start_token, end_token):
    del start_token, end_token
    pe = _positional_encoding(L, D)
    out = _emb_pe(table.reshape(V * D), x.reshape(NW, NCHUNK, C), pe)
    return out.reshape(B, L, D)


# speedup vs baseline: 1.2428x; 1.2428x over previous
import functools

import jax
import jax.numpy as jnp
from jax import lax
from jax.experimental import pallas as pl
from jax.experimental.pallas import tpu as pltpu
from jax.experimental.pallas import tpu_sc as plsc

B = 128
L = 512
D = 512
V = 42
N = B * L
NC, NS = 2, 16
NW = NC * NS
TPW = N // NW
C = 32
NCHUNK = TPW // C
NBODY = NCHUNK // 2
LANES = 16
G = D // LANES


def _positional_encoding(max_len, d_model):
    even_i = jnp.arange(0, d_model, 2).astype(jnp.float32)
    denominator = jnp.power(10000.0, even_i / d_model)
    position = jnp.arange(max_len, dtype=jnp.float32).reshape(max_len, 1)
    even_pe = jnp.sin(position / denominator)
    odd_pe = jnp.cos(position / denominator)
    return jnp.stack([even_pe, odd_pe], axis=2).reshape(max_len, d_model)


@functools.partial(
    pl.kernel,
    mesh=plsc.VectorSubcoreMesh(core_axis_name="c", subcore_axis_name="s"),
    out_type=jax.ShapeDtypeStruct((N, D), jnp.float32),
    scratch_types=[
        pltpu.VMEM((V * D,), jnp.float32),
        pltpu.VMEM((C,), jnp.int32),
        pltpu.VMEM((C,), jnp.int32),
        pltpu.VMEM((C, D), jnp.float32),
        pltpu.VMEM((C, D), jnp.float32),
        pltpu.VMEM((C, D), jnp.float32),
        pltpu.VMEM((C, D), jnp.float32),
        pltpu.SemaphoreType.DMA,
        pltpu.SemaphoreType.DMA,
        pltpu.SemaphoreType.DMA,
        pltpu.SemaphoreType.DMA,
        pltpu.SemaphoreType.DMA,
        pltpu.SemaphoreType.DMA,
    ],
)
def _emb_pe(table_hbm, idx_hbm, pe_hbm, out_hbm,
            table_v, ids0, ids1, pe0, pe1, rows0, rows1,
            si0, si1, sp0, sp1, so0, so1):
    wid = lax.axis_index("s") * NC + lax.axis_index("c")
    tok_base = wid * TPW

    ids = (ids0, ids1)
    pes = (pe0, pe1)
    rows = (rows0, rows1)
    si = (si0, si1)
    sp = (sp0, sp1)
    so = (so0, so1)

    pltpu.sync_copy(table_hbm, table_v)

    def issue_idx(c, p):
        return pltpu.async_copy(idx_hbm.at[wid, c], ids[p], si[p])

    def wait_idx(p):
        pltpu.make_async_copy(idx_hbm.at[wid, 0], ids[p], si[p]).wait()

    def issue_pe(c, p):
        l0 = (c % (L // C)) * C
        return pltpu.async_copy(pe_hbm.at[pl.ds(l0, C)], pes[p], sp[p])

    def wait_pe(p):
        pltpu.make_async_copy(pe_hbm.at[pl.ds(0, C)], pes[p], sp[p]).wait()

    def compute_rows(p):
        r, q, s = rows[p], pes[p], ids[p]

        def blk16(i, acc):
            t0 = i * LANES
            offvec = s[pl.ds(t0, LANES)] * D
            bases = [offvec[j] for j in range(LANES)]
            for j in range(LANES):
                base = bases[j]
                t = t0 + j
                for g0 in range(0, G, 8):
                    offs = [(g0 + jj) * LANES for jj in range(8)]
                    vals = [
                        table_v[pl.ds(base + o, LANES)] + q[t, pl.ds(o, LANES)]
                        for o in offs
                    ]
                    for o, val in zip(offs, vals):
                        r[t, pl.ds(o, LANES)] = val
            return acc

        lax.fori_loop(0, C // LANES, blk16, 0)

    def issue_out(c, p):
        return pltpu.async_copy(
            rows[p], out_hbm.at[pl.ds(tok_base + c * C, C)], so[p]
        )

    def wait_out(p):
        pltpu.make_async_copy(
            rows[p], out_hbm.at[pl.ds(tok_base, C)], so[p]
        ).wait()

    issue_idx(0, 0)
    issue_pe(0, 0)
    issue_idx(1, 1)
    issue_pe(1, 1)

    def body(k, carry):
        c0 = 2 * k
        c1 = 2 * k + 1

        wait_idx(0)
        wait_pe(0)

        @pl.when(k > 0)
        def _():
            wait_out(0)

        compute_rows(0)

        @pl.when(k < NBODY - 1)
        def _():
            issue_idx(c0 + 2, 0)
            issue_pe(c0 + 2, 0)

        issue_out(c0, 0)

        wait_idx(1)
        wait_pe(1)

        @pl.when(k > 0)
        def _():
            wait_out(1)

        compute_rows(1)

        @pl.when(k < NBODY - 1)
        def _():
            issue_idx(c1 + 2, 1)
            issue_pe(c1 + 2, 1)

        issue_out(c1, 1)
        return carry

    lax.fori_loop(0, NBODY, body, 0)
    wait_out(0)
    wait_out(1)


def kernel(x, table, start_token, end_token):
    del start_token, end_token
    pe = _positional_encoding(L, D)
    out = _emb_pe(table.reshape(V * D), x.reshape(NW, NCHUNK, C), pe)
    return out.reshape(B, L, D)


# speedup vs baseline: 2.4758x; 1.9921x over previous
import functools

import jax
import jax.numpy as jnp
from jax import lax
from jax.experimental import pallas as pl
from jax.experimental.pallas import tpu as pltpu
from jax.experimental.pallas import tpu_sc as plsc

B = 128
L = 512
D = 512
V = 42
N = B * L
NC, NS = 2, 16
NW = NC * NS
TPW = N // NW
C = 32
NCHUNK = TPW // C
NBODY = NCHUNK // 2
LANES = 16
G = D // LANES


def _positional_encoding(max_len, d_model):
    even_i = jnp.arange(0, d_model, 2).astype(jnp.float32)
    denominator = jnp.power(10000.0, even_i / d_model)
    position = jnp.arange(max_len, dtype=jnp.float32).reshape(max_len, 1)
    even_pe = jnp.sin(position / denominator)
    odd_pe = jnp.cos(position / denominator)
    return jnp.stack([even_pe, odd_pe], axis=2).reshape(max_len, d_model)


@functools.partial(
    pl.kernel,
    mesh=plsc.VectorSubcoreMesh(core_axis_name="c", subcore_axis_name="s"),
    out_type=jax.ShapeDtypeStruct((N, D), jnp.float32),
    scratch_types=[
        pltpu.VMEM((V * D,), jnp.float32),
        pltpu.VMEM((C,), jnp.int32),
        pltpu.VMEM((C,), jnp.int32),
        pltpu.VMEM((C, D), jnp.float32),
        pltpu.VMEM((C, D), jnp.float32),
        pltpu.VMEM((C, D), jnp.float32),
        pltpu.VMEM((C, D), jnp.float32),
        pltpu.SemaphoreType.DMA,
        pltpu.SemaphoreType.DMA,
        pltpu.SemaphoreType.DMA,
        pltpu.SemaphoreType.DMA,
        pltpu.SemaphoreType.DMA,
        pltpu.SemaphoreType.DMA,
    ],
)
def _emb_pe(table_hbm, idx_hbm, pe_hbm, out_hbm,
            table_v, ids0, ids1, pe0, pe1, rows0, rows1,
            si0, si1, sp0, sp1, so0, so1):
    wid = lax.axis_index("s") * NC + lax.axis_index("c")
    tok_base = wid * TPW

    ids = (ids0, ids1)
    pes = (pe0, pe1)
    rows = (rows0, rows1)
    si = (si0, si1)
    sp = (sp0, sp1)
    so = (so0, so1)

    pltpu.sync_copy(table_hbm, table_v)

    def issue_idx(c, p):
        return pltpu.async_copy(idx_hbm.at[wid, c], ids[p], si[p])

    def wait_idx(p):
        pltpu.make_async_copy(idx_hbm.at[wid, 0], ids[p], si[p]).wait()

    def issue_pe(c, p):
        l0 = (c % (L // C)) * C
        return pltpu.async_copy(pe_hbm.at[pl.ds(l0, C)], pes[p], sp[p])

    def wait_pe(p):
        pltpu.make_async_copy(pe_hbm.at[pl.ds(0, C)], pes[p], sp[p]).wait()

    def compute_rows(p):
        r, q, s = rows[p], pes[p], ids[p]

        def blk16(i, acc):
            t0 = i * LANES
            offvec = s[pl.ds(t0, LANES)] * D
            bases = [offvec[j] for j in range(LANES)]
            for j in range(LANES):
                base = bases[j]
                t = t0 + j
                for g0 in range(0, G, 8):
                    offs = [(g0 + jj) * LANES for jj in range(8)]
                    vals = [
                        table_v[pl.ds(base + o, LANES)] + q[t, pl.ds(o, LANES)]
                        for o in offs
                    ]
                    for o, val in zip(offs, vals):
                        r[t, pl.ds(o, LANES)] = val
            return acc

        lax.fori_loop(0, C // LANES, blk16, 0)

    def issue_out(c, p):
        return pltpu.async_copy(
            rows[p], out_hbm.at[pl.ds(tok_base + c * C, C)], so[p]
        )

    def wait_out(p):
        pltpu.make_async_copy(
            rows[p], out_hbm.at[pl.ds(tok_base, C)], so[p]
        ).wait()

    issue_idx(0, 0)
    issue_pe(0, 0)
    issue_idx(1, 1)
    issue_pe(1, 1)

    def body(k, carry):
        c0 = 2 * k
        c1 = 2 * k + 1

        wait_idx(0)
        wait_pe(0)

        @pl.when(k > 0)
        def _():
            wait_out(0)

        compute_rows(0)

        @pl.when(k < NBODY - 1)
        def _():
            issue_idx(c0 + 2, 0)
            issue_pe(c0 + 2, 0)

        issue_out(c0, 0)

        wait_idx(1)
        wait_pe(1)

        @pl.when(k > 0)
        def _():
            wait_out(1)

        compute_rows(1)

        @pl.when(k < NBODY - 1)
        def _():
            issue_idx(c1 + 2, 1)
            issue_pe(c1 + 2, 1)

        issue_out(c1, 1)
        return carry

    lax.fori_loop(0, NBODY, body, 0)
    wait_out(0)
    wait_out(1)


VPAD = 64


def _emb_pe_tc_block(x_ref, table_ref, pe_ref, out_ref):
    xb = x_ref[0, 0, :]
    iota_v = lax.broadcasted_iota(jnp.int32, (L, VPAD), 1)
    onehot = (xb[:, None] == iota_v).astype(jnp.float32)
    emb = jnp.dot(onehot, table_ref[...],
                  preferred_element_type=jnp.float32)
    out_ref[0] = emb + pe_ref[...]


def _emb_pe_tc(x3, table_pad, pe, nb):
    return pl.pallas_call(
        _emb_pe_tc_block,
        grid=(nb,),
        in_specs=[
            pl.BlockSpec((1, 1, L), lambda i: (i, 0, 0)),
            pl.BlockSpec((VPAD, D), lambda i: (0, 0)),
            pl.BlockSpec((L, D), lambda i: (0, 0)),
        ],
        out_specs=pl.BlockSpec((1, L, D), lambda i: (i, 0, 0)),
        out_shape=jax.ShapeDtypeStruct((nb, L, D), jnp.float32),
    )(x3, table_pad, pe)


def kernel(x, table, start_token, end_token):
    del start_token, end_token
    pe = _positional_encoding(L, D)
    table_pad = jnp.zeros((VPAD, D), jnp.float32).at[:V].set(table)
    out = _emb_pe_tc(x.reshape(B, 1, L), table_pad, pe, B)
    return out.reshape(B, L, D)
